# baseline (speedup 1.0000x reference)
import functools

import jax
import jax.numpy as jnp
from jax import lax
from jax.experimental import pallas as pl
from jax.experimental.pallas import tpu as pltpu
from jax.experimental.pallas import tpu_sc as plsc

NC = 2
NS = 16
NW = NC * NS
LANES = 16
TC_T = 160
TC_TB = 8
TC_BB = 512
SC_BB = 512
NBUF = 2


def _sc_decode_call(xT2, alphabet_codes, V, B, t0, sc_t):
    nchunks = sc_t * (B // SC_BB)
    per_w = nchunks // NW

    mesh = plsc.VectorSubcoreMesh(
        core_axis_name="c", subcore_axis_name="s",
        num_cores=NC, num_subcores=NS)

    @functools.partial(
        pl.kernel,
        out_type=jax.ShapeDtypeStruct((sc_t * B,), jnp.int32),
        mesh=mesh,
        scratch_types=(
            [pltpu.VMEM((V, SC_BB), jnp.float32) for _ in range(NBUF)]
            + [pltpu.VMEM((SC_BB,), jnp.int32),
               pltpu.VMEM((V,), jnp.int32)]
            + [pltpu.SemaphoreType.DMA for _ in range(NBUF)]
        ),
        compiler_params=pltpu.CompilerParams(
            needs_layout_passes=False, use_tc_tiling_on_sc=True),
    )
    def sc_decode(x_hbm, alpha_hbm, out_hbm, *refs):
        bufs = refs[:NBUF]
        obuf, alpha_v = refs[NBUF], refs[NBUF + 1]
        sems = refs[NBUF + 2:]

        wid = lax.axis_index("s") * NC + lax.axis_index("c")
        k0 = wid * per_w
        ncol = B // SC_BB

        def in_slice(k):
            t = t0 + k // ncol
            b0 = (k % ncol) * SC_BB
            return x_hbm.at[pl.ds(t * V, V), pl.ds(b0, SC_BB)]

        pltpu.sync_copy(alpha_hbm, alpha_v)
        for b in range(NBUF):
            pltpu.async_copy(in_slice(k0 + b), bufs[b], sems[b])

        def chunk_body(k, b):
            pltpu.make_async_copy(in_slice(k), bufs[b], sems[b]).wait()
            bb = bufs[b]

            def group(gr, carry):
                c0 = gr * LANES
                ms, idxs = [], []
                for j in range(8):
                    base_c = j * 8
                    m = bb[base_c, pl.ds(c0, LANES)]
                    idx = jnp.full((LANES,), base_c, jnp.int32)
                    for q in range(1, 8):
                        c = base_c + q
                        v = bb[c, pl.ds(c0, LANES)]
                        upd = v > m
                        m = jnp.where(upd, v, m)
                        idx = jnp.where(upd, c, idx)
                    ms.append(m)
                    idxs.append(idx)
                while len(ms) > 1:
                    nm, ni = [], []
                    for j in range(0, len(ms), 2):
                        upd = ms[j + 1] > ms[j]
                        nm.append(jnp.where(upd, ms[j + 1], ms[j]))
                        ni.append(jnp.where(upd, idxs[j + 1], idxs[j]))
                    ms, idxs = nm, ni
                obuf[pl.ds(c0, LANES)] = plsc.load_gather(alpha_v, [idxs[0]])
                return carry

            lax.fori_loop(0, SC_BB // LANES, group, 0)

            t = t0 + k // ncol
            b0 = (k % ncol) * SC_BB
            pltpu.sync_copy(
                obuf, out_hbm.at[pl.ds((t - t0) * B + b0, SC_BB)])

            nxt = k + NBUF

            @pl.when(nxt < k0 + per_w)
            def _():
                pltpu.async_copy(in_slice(nxt), bufs[b], sems[b])

        def ring_body(i, carry):
            for b in range(NBUF):
                chunk_body(k0 + i * NBUF + b, b)
            return carry

        lax.fori_loop(0, per_w // NBUF, ring_body, 0)

    return sc_decode(xT2, alphabet_codes)


def _tc_decode_call(xT, comb2, T, V, B):
    def tc_kernel(comb_ref, x_ref, o_ref):
        xb = x_ref[...]
        m = jnp.max(xb, axis=1, keepdims=True)
        keyf = jnp.where(xb == m, comb_ref[...][None, :, :], -jnp.inf)
        best = jnp.max(keyf, axis=1)
        o_ref[...] = (-best).astype(jnp.int32) & 255

    return pl.pallas_call(
        tc_kernel,
        grid=(TC_T // TC_TB, B // TC_BB),
        in_specs=[
            pl.BlockSpec((V, 1), lambda i, j: (0, 0)),
            pl.BlockSpec((TC_TB, V, TC_BB), lambda i, j: (i, 0, j)),
        ],
        out_specs=pl.BlockSpec((TC_TB, TC_BB), lambda i, j: (i, j)),
        out_shape=jax.ShapeDtypeStruct((TC_T, B), jnp.int32),
        compiler_params=pltpu.CompilerParams(
            dimension_semantics=("parallel", "parallel")),
    )(comb2, xT)


def kernel(x, alphabet_codes):
    B, T, V = x.shape
    sc_t = T - TC_T
    xT = jnp.transpose(x, (1, 2, 0))
    xT2 = xT.reshape(T * V, B)
    comb2 = -((jnp.arange(V, dtype=jnp.int32) << 8) | alphabet_codes
              ).astype(jnp.float32).reshape(V, 1)

    sc_out = _sc_decode_call(xT2, alphabet_codes, V, B, TC_T, sc_t)
    tc_out = _tc_decode_call(xT, comb2, T, V, B)
    oT = jnp.concatenate([tc_out, sc_out.reshape(sc_t, B)], axis=0)
    return jnp.transpose(oT)


# speedup vs baseline: 1.0724x; 1.0724x over previous
import functools

import jax
import jax.numpy as jnp
from jax import lax
from jax.experimental import pallas as pl
from jax.experimental.pallas import tpu as pltpu
from jax.experimental.pallas import tpu_sc as plsc

NC = 2
NS = 16
NW = NC * NS
LANES = 16
TC_T = 144
TC_TB = 8
TC_BB = 512
SC_BB = 512
NBUF = 2


def _sc_decode_call(xT2, alphabet_codes, V, B, t0, sc_t):
    nchunks = sc_t * (B // SC_BB)
    per_w = nchunks // NW

    mesh = plsc.VectorSubcoreMesh(
        core_axis_name="c", subcore_axis_name="s",
        num_cores=NC, num_subcores=NS)

    @functools.partial(
        pl.kernel,
        out_type=jax.ShapeDtypeStruct((sc_t * B,), jnp.int32),
        mesh=mesh,
        scratch_types=(
            [pltpu.VMEM((V, SC_BB), jnp.float32) for _ in range(NBUF)]
            + [pltpu.VMEM((SC_BB,), jnp.int32),
               pltpu.VMEM((V,), jnp.int32)]
            + [pltpu.SemaphoreType.DMA for _ in range(NBUF)]
        ),
        compiler_params=pltpu.CompilerParams(
            needs_layout_passes=False, use_tc_tiling_on_sc=True),
    )
    def sc_decode(x_hbm, alpha_hbm, out_hbm, *refs):
        bufs = refs[:NBUF]
        obuf, alpha_v = refs[NBUF], refs[NBUF + 1]
        sems = refs[NBUF + 2:]

        wid = lax.axis_index("s") * NC + lax.axis_index("c")
        k0 = wid * per_w
        ncol = B // SC_BB

        def in_slice(k):
            t = t0 + k // ncol
            b0 = (k % ncol) * SC_BB
            return x_hbm.at[pl.ds(t * V, V), pl.ds(b0, SC_BB)]

        pltpu.sync_copy(alpha_hbm, alpha_v)
        for b in range(NBUF):
            pltpu.async_copy(in_slice(k0 + b), bufs[b], sems[b])

        def chunk_body(k, b):
            pltpu.make_async_copy(in_slice(k), bufs[b], sems[b]).wait()
            bb = bufs[b]

            def group(gr, carry):
                c0 = gr * LANES
                ms, idxs = [], []
                for j in range(8):
                    base_c = j * 8
                    m = bb[base_c, pl.ds(c0, LANES)]
                    idx = jnp.full((LANES,), base_c, jnp.int32)
                    for q in range(1, 8):
                        c = base_c + q
                        v = bb[c, pl.ds(c0, LANES)]
                        upd = v > m
                        m = jnp.where(upd, v, m)
                        idx = jnp.where(upd, c, idx)
                    ms.append(m)
                    idxs.append(idx)
                while len(ms) > 1:
                    nm, ni = [], []
                    for j in range(0, len(ms), 2):
                        upd = ms[j + 1] > ms[j]
                        nm.append(jnp.where(upd, ms[j + 1], ms[j]))
                        ni.append(jnp.where(upd, idxs[j + 1], idxs[j]))
                    ms, idxs = nm, ni
                obuf[pl.ds(c0, LANES)] = plsc.load_gather(alpha_v, [idxs[0]])
                return carry

            lax.fori_loop(0, SC_BB // LANES, group, 0)

            t = t0 + k // ncol
            b0 = (k % ncol) * SC_BB
            pltpu.sync_copy(
                obuf, out_hbm.at[pl.ds((t - t0) * B + b0, SC_BB)])

            nxt = k + NBUF

            @pl.when(nxt < k0 + per_w)
            def _():
                pltpu.async_copy(in_slice(nxt), bufs[b], sems[b])

        def ring_body(i, carry):
            for b in range(NBUF):
                chunk_body(k0 + i * NBUF + b, b)
            return carry

        lax.fori_loop(0, per_w // NBUF, ring_body, 0)

    return sc_decode(xT2, alphabet_codes)


def _tc_decode_call(xT, comb2, T, V, B):
    def tc_kernel(comb_ref, x_ref, o_ref):
        xb = x_ref[...]
        m = jnp.max(xb, axis=1, keepdims=True)
        keyf = jnp.where(xb == m, comb_ref[...][None, :, :], -jnp.inf)
        best = jnp.max(keyf, axis=1)
        o_ref[...] = (-best).astype(jnp.int32) & 255

    return pl.pallas_call(
        tc_kernel,
        grid=(TC_T // TC_TB, B // TC_BB),
        in_specs=[
            pl.BlockSpec((V, 1), lambda i, j: (0, 0)),
            pl.BlockSpec((TC_TB, V, TC_BB), lambda i, j: (i, 0, j)),
        ],
        out_specs=pl.BlockSpec((TC_TB, TC_BB), lambda i, j: (i, j)),
        out_shape=jax.ShapeDtypeStruct((TC_T, B), jnp.int32),
        compiler_params=pltpu.CompilerParams(
            dimension_semantics=("parallel", "parallel")),
    )(comb2, xT)


def kernel(x, alphabet_codes):
    B, T, V = x.shape
    sc_t = T - TC_T
    xT = jnp.transpose(x, (1, 2, 0))
    xT2 = xT.reshape(T * V, B)
    comb2 = -((jnp.arange(V, dtype=jnp.int32) << 8) | alphabet_codes
              ).astype(jnp.float32).reshape(V, 1)

    sc_out = _sc_decode_call(xT2, alphabet_codes, V, B, TC_T, sc_t)
    tc_out = _tc_decode_call(xT, comb2, T, V, B)
    oT = jnp.concatenate([tc_out, sc_out.reshape(sc_t, B)], axis=0)
    return jnp.transpose(oT)


# speedup vs baseline: 1.1185x; 1.0430x over previous
import functools

import jax
import jax.numpy as jnp
from jax import lax
from jax.experimental import pallas as pl
from jax.experimental.pallas import tpu as pltpu
from jax.experimental.pallas import tpu_sc as plsc

NC = 2
NS = 16
NW = NC * NS
LANES = 16
TC_T = 136
TC_TB = 8
TC_BB = 512
SC_BB = 512
NBUF = 2


def _sc_decode_call(xT2, alphabet_codes, V, B, t0, sc_t):
    nchunks = sc_t * (B // SC_BB)
    per_w = nchunks // NW

    mesh = plsc.VectorSubcoreMesh(
        core_axis_name="c", subcore_axis_name="s",
        num_cores=NC, num_subcores=NS)

    @functools.partial(
        pl.kernel,
        out_type=jax.ShapeDtypeStruct((sc_t * B,), jnp.int32),
        mesh=mesh,
        scratch_types=(
            [pltpu.VMEM((V, SC_BB), jnp.float32) for _ in range(NBUF)]
            + [pltpu.VMEM((SC_BB,), jnp.int32),
               pltpu.VMEM((V,), jnp.int32)]
            + [pltpu.SemaphoreType.DMA for _ in range(NBUF)]
        ),
        compiler_params=pltpu.CompilerParams(
            needs_layout_passes=False, use_tc_tiling_on_sc=True),
    )
    def sc_decode(x_hbm, alpha_hbm, out_hbm, *refs):
        bufs = refs[:NBUF]
        obuf, alpha_v = refs[NBUF], refs[NBUF + 1]
        sems = refs[NBUF + 2:]

        wid = lax.axis_index("s") * NC + lax.axis_index("c")
        k0 = wid * per_w
        ncol = B // SC_BB

        def in_slice(k):
            t = t0 + k // ncol
            b0 = (k % ncol) * SC_BB
            return x_hbm.at[pl.ds(t * V, V), pl.ds(b0, SC_BB)]

        pltpu.sync_copy(alpha_hbm, alpha_v)
        for b in range(NBUF):
            pltpu.async_copy(in_slice(k0 + b), bufs[b], sems[b])

        def chunk_body(k, b):
            pltpu.make_async_copy(in_slice(k), bufs[b], sems[b]).wait()
            bb = bufs[b]

            def group(gr, carry):
                c0 = gr * LANES
                ms, idxs = [], []
                for j in range(8):
                    base_c = j * 8
                    m = bb[base_c, pl.ds(c0, LANES)]
                    idx = jnp.full((LANES,), base_c, jnp.int32)
                    for q in range(1, 8):
                        c = base_c + q
                        v = bb[c, pl.ds(c0, LANES)]
                        upd = v > m
                        m = jnp.where(upd, v, m)
                        idx = jnp.where(upd, c, idx)
                    ms.append(m)
                    idxs.append(idx)
                while len(ms) > 1:
                    nm, ni = [], []
                    for j in range(0, len(ms), 2):
                        upd = ms[j + 1] > ms[j]
                        nm.append(jnp.where(upd, ms[j + 1], ms[j]))
                        ni.append(jnp.where(upd, idxs[j + 1], idxs[j]))
                    ms, idxs = nm, ni
                obuf[pl.ds(c0, LANES)] = plsc.load_gather(alpha_v, [idxs[0]])
                return carry

            lax.fori_loop(0, SC_BB // LANES, group, 0)

            t = t0 + k // ncol
            b0 = (k % ncol) * SC_BB
            pltpu.sync_copy(
                obuf, out_hbm.at[pl.ds((t - t0) * B + b0, SC_BB)])

            nxt = k + NBUF

            @pl.when(nxt < k0 + per_w)
            def _():
                pltpu.async_copy(in_slice(nxt), bufs[b], sems[b])

        def ring_body(i, carry):
            for b in range(NBUF):
                chunk_body(k0 + i * NBUF + b, b)
            return carry

        lax.fori_loop(0, per_w // NBUF, ring_body, 0)

    return sc_decode(xT2, alphabet_codes)


def _tc_decode_call(xT, comb2, T, V, B):
    def tc_kernel(comb_ref, x_ref, o_ref):
        xb = x_ref[...]
        m = jnp.max(xb, axis=1, keepdims=True)
        keyf = jnp.where(xb == m, comb_ref[...][None, :, :], -jnp.inf)
        best = jnp.max(keyf, axis=1)
        o_ref[...] = (-best).astype(jnp.int32) & 255

    return pl.pallas_call(
        tc_kernel,
        grid=(TC_T // TC_TB, B // TC_BB),
        in_specs=[
            pl.BlockSpec((V, 1), lambda i, j: (0, 0)),
            pl.BlockSpec((TC_TB, V, TC_BB), lambda i, j: (i, 0, j)),
        ],
        out_specs=pl.BlockSpec((TC_TB, TC_BB), lambda i, j: (i, j)),
        out_shape=jax.ShapeDtypeStruct((TC_T, B), jnp.int32),
        compiler_params=pltpu.CompilerParams(
            dimension_semantics=("parallel", "parallel")),
    )(comb2, xT)


def kernel(x, alphabet_codes):
    B, T, V = x.shape
    sc_t = T - TC_T
    xT = jnp.transpose(x, (1, 2, 0))
    xT2 = xT.reshape(T * V, B)
    comb2 = -((jnp.arange(V, dtype=jnp.int32) << 8) | alphabet_codes
              ).astype(jnp.float32).reshape(V, 1)

    sc_out = _sc_decode_call(xT2, alphabet_codes, V, B, TC_T, sc_t)
    tc_out = _tc_decode_call(xT, comb2, T, V, B)
    oT = jnp.concatenate([tc_out, sc_out.reshape(sc_t, B)], axis=0)
    return jnp.transpose(oT)


# speedup vs baseline: 1.1656x; 1.0421x over previous
import functools

import jax
import jax.numpy as jnp
from jax import lax
from jax.experimental import pallas as pl
from jax.experimental.pallas import tpu as pltpu
from jax.experimental.pallas import tpu_sc as plsc

NC = 2
NS = 16
NW = NC * NS
LANES = 16
TC_T = 128
TC_TB = 8
TC_BB = 512
SC_BB = 512
NBUF = 2


def _sc_decode_call(xT2, alphabet_codes, V, B, t0, sc_t):
    nchunks = sc_t * (B // SC_BB)
    per_w = nchunks // NW

    mesh = plsc.VectorSubcoreMesh(
        core_axis_name="c", subcore_axis_name="s",
        num_cores=NC, num_subcores=NS)

    @functools.partial(
        pl.kernel,
        out_type=jax.ShapeDtypeStruct((sc_t * B,), jnp.int32),
        mesh=mesh,
        scratch_types=(
            [pltpu.VMEM((V, SC_BB), jnp.float32) for _ in range(NBUF)]
            + [pltpu.VMEM((SC_BB,), jnp.int32),
               pltpu.VMEM((V,), jnp.int32)]
            + [pltpu.SemaphoreType.DMA for _ in range(NBUF)]
        ),
        compiler_params=pltpu.CompilerParams(
            needs_layout_passes=False, use_tc_tiling_on_sc=True),
    )
    def sc_decode(x_hbm, alpha_hbm, out_hbm, *refs):
        bufs = refs[:NBUF]
        obuf, alpha_v = refs[NBUF], refs[NBUF + 1]
        sems = refs[NBUF + 2:]

        wid = lax.axis_index("s") * NC + lax.axis_index("c")
        k0 = wid * per_w
        ncol = B // SC_BB

        def in_slice(k):
            t = t0 + k // ncol
            b0 = (k % ncol) * SC_BB
            return x_hbm.at[pl.ds(t * V, V), pl.ds(b0, SC_BB)]

        pltpu.sync_copy(alpha_hbm, alpha_v)
        for b in range(NBUF):
            pltpu.async_copy(in_slice(k0 + b), bufs[b], sems[b])

        def chunk_body(k, b):
            pltpu.make_async_copy(in_slice(k), bufs[b], sems[b]).wait()
            bb = bufs[b]

            def group(gr, carry):
                c0 = gr * LANES
                ms, idxs = [], []
                for j in range(8):
                    base_c = j * 8
                    m = bb[base_c, pl.ds(c0, LANES)]
                    idx = jnp.full((LANES,), base_c, jnp.int32)
                    for q in range(1, 8):
                        c = base_c + q
                        v = bb[c, pl.ds(c0, LANES)]
                        upd = v > m
                        m = jnp.where(upd, v, m)
                        idx = jnp.where(upd, c, idx)
                    ms.append(m)
                    idxs.append(idx)
                while len(ms) > 1:
                    nm, ni = [], []
                    for j in range(0, len(ms), 2):
                        upd = ms[j + 1] > ms[j]
                        nm.append(jnp.where(upd, ms[j + 1], ms[j]))
                        ni.append(jnp.where(upd, idxs[j + 1], idxs[j]))
                    ms, idxs = nm, ni
                obuf[pl.ds(c0, LANES)] = plsc.load_gather(alpha_v, [idxs[0]])
                return carry

            lax.fori_loop(0, SC_BB // LANES, group, 0)

            t = t0 + k // ncol
            b0 = (k % ncol) * SC_BB
            pltpu.sync_copy(
                obuf, out_hbm.at[pl.ds((t - t0) * B + b0, SC_BB)])

            nxt = k + NBUF

            @pl.when(nxt < k0 + per_w)
            def _():
                pltpu.async_copy(in_slice(nxt), bufs[b], sems[b])

        def ring_body(i, carry):
            for b in range(NBUF):
                chunk_body(k0 + i * NBUF + b, b)
            return carry

        lax.fori_loop(0, per_w // NBUF, ring_body, 0)

    return sc_decode(xT2, alphabet_codes)


def _tc_decode_call(xT, comb2, T, V, B):
    def tc_kernel(comb_ref, x_ref, o_ref):
        xb = x_ref[...]
        m = jnp.max(xb, axis=1, keepdims=True)
        keyf = jnp.where(xb == m, comb_ref[...][None, :, :], -jnp.inf)
        best = jnp.max(keyf, axis=1)
        o_ref[...] = (-best).astype(jnp.int32) & 255

    return pl.pallas_call(
        tc_kernel,
        grid=(TC_T // TC_TB, B // TC_BB),
        in_specs=[
            pl.BlockSpec((V, 1), lambda i, j: (0, 0)),
            pl.BlockSpec((TC_TB, V, TC_BB), lambda i, j: (i, 0, j)),
        ],
        out_specs=pl.BlockSpec((TC_TB, TC_BB), lambda i, j: (i, j)),
        out_shape=jax.ShapeDtypeStruct((TC_T, B), jnp.int32),
        compiler_params=pltpu.CompilerParams(
            dimension_semantics=("parallel", "parallel")),
    )(comb2, xT)


def kernel(x, alphabet_codes):
    B, T, V = x.shape
    sc_t = T - TC_T
    xT = jnp.transpose(x, (1, 2, 0))
    xT2 = xT.reshape(T * V, B)
    comb2 = -((jnp.arange(V, dtype=jnp.int32) << 8) | alphabet_codes
              ).astype(jnp.float32).reshape(V, 1)

    sc_out = _sc_decode_call(xT2, alphabet_codes, V, B, TC_T, sc_t)
    tc_out = _tc_decode_call(xT, comb2, T, V, B)
    oT = jnp.concatenate([tc_out, sc_out.reshape(sc_t, B)], axis=0)
    return jnp.transpose(oT)


# speedup vs baseline: 1.2167x; 1.0439x over previous
import functools

import jax
import jax.numpy as jnp
from jax import lax
from jax.experimental import pallas as pl
from jax.experimental.pallas import tpu as pltpu
from jax.experimental.pallas import tpu_sc as plsc

NC = 2
NS = 16
NW = NC * NS
LANES = 16
TC_T = 120
TC_TB = 8
TC_BB = 512
SC_BB = 512
NBUF = 2


def _sc_decode_call(xT2, alphabet_codes, V, B, t0, sc_t):
    nchunks = sc_t * (B // SC_BB)
    per_w = nchunks // NW

    mesh = plsc.VectorSubcoreMesh(
        core_axis_name="c", subcore_axis_name="s",
        num_cores=NC, num_subcores=NS)

    @functools.partial(
        pl.kernel,
        out_type=jax.ShapeDtypeStruct((sc_t * B,), jnp.int32),
        mesh=mesh,
        scratch_types=(
            [pltpu.VMEM((V, SC_BB), jnp.float32) for _ in range(NBUF)]
            + [pltpu.VMEM((SC_BB,), jnp.int32),
               pltpu.VMEM((V,), jnp.int32)]
            + [pltpu.SemaphoreType.DMA for _ in range(NBUF)]
        ),
        compiler_params=pltpu.CompilerParams(
            needs_layout_passes=False, use_tc_tiling_on_sc=True),
    )
    def sc_decode(x_hbm, alpha_hbm, out_hbm, *refs):
        bufs = refs[:NBUF]
        obuf, alpha_v = refs[NBUF], refs[NBUF + 1]
        sems = refs[NBUF + 2:]

        wid = lax.axis_index("s") * NC + lax.axis_index("c")
        k0 = wid * per_w
        ncol = B // SC_BB

        def in_slice(k):
            t = t0 + k // ncol
            b0 = (k % ncol) * SC_BB
            return x_hbm.at[pl.ds(t * V, V), pl.ds(b0, SC_BB)]

        pltpu.sync_copy(alpha_hbm, alpha_v)
        for b in range(NBUF):
            pltpu.async_copy(in_slice(k0 + b), bufs[b], sems[b])

        def chunk_body(k, b):
            pltpu.make_async_copy(in_slice(k), bufs[b], sems[b]).wait()
            bb = bufs[b]

            def group(gr, carry):
                c0 = gr * LANES
                ms, idxs = [], []
                for j in range(8):
                    base_c = j * 8
                    m = bb[base_c, pl.ds(c0, LANES)]
                    idx = jnp.full((LANES,), base_c, jnp.int32)
                    for q in range(1, 8):
                        c = base_c + q
                        v = bb[c, pl.ds(c0, LANES)]
                        upd = v > m
                        m = jnp.where(upd, v, m)
                        idx = jnp.where(upd, c, idx)
                    ms.append(m)
                    idxs.append(idx)
                while len(ms) > 1:
                    nm, ni = [], []
                    for j in range(0, len(ms), 2):
                        upd = ms[j + 1] > ms[j]
                        nm.append(jnp.where(upd, ms[j + 1], ms[j]))
                        ni.append(jnp.where(upd, idxs[j + 1], idxs[j]))
                    ms, idxs = nm, ni
                obuf[pl.ds(c0, LANES)] = plsc.load_gather(alpha_v, [idxs[0]])
                return carry

            lax.fori_loop(0, SC_BB // LANES, group, 0)

            t = t0 + k // ncol
            b0 = (k % ncol) * SC_BB
            pltpu.sync_copy(
                obuf, out_hbm.at[pl.ds((t - t0) * B + b0, SC_BB)])

            nxt = k + NBUF

            @pl.when(nxt < k0 + per_w)
            def _():
                pltpu.async_copy(in_slice(nxt), bufs[b], sems[b])

        def ring_body(i, carry):
            for b in range(NBUF):
                chunk_body(k0 + i * NBUF + b, b)
            return carry

        lax.fori_loop(0, per_w // NBUF, ring_body, 0)

    return sc_decode(xT2, alphabet_codes)


def _tc_decode_call(xT, comb2, T, V, B):
    def tc_kernel(comb_ref, x_ref, o_ref):
        xb = x_ref[...]
        m = jnp.max(xb, axis=1, keepdims=True)
        keyf = jnp.where(xb == m, comb_ref[...][None, :, :], -jnp.inf)
        best = jnp.max(keyf, axis=1)
        o_ref[...] = (-best).astype(jnp.int32) & 255

    return pl.pallas_call(
        tc_kernel,
        grid=(TC_T // TC_TB, B // TC_BB),
        in_specs=[
            pl.BlockSpec((V, 1), lambda i, j: (0, 0)),
            pl.BlockSpec((TC_TB, V, TC_BB), lambda i, j: (i, 0, j)),
        ],
        out_specs=pl.BlockSpec((TC_TB, TC_BB), lambda i, j: (i, j)),
        out_shape=jax.ShapeDtypeStruct((TC_T, B), jnp.int32),
        compiler_params=pltpu.CompilerParams(
            dimension_semantics=("parallel", "parallel")),
    )(comb2, xT)


def kernel(x, alphabet_codes):
    B, T, V = x.shape
    sc_t = T - TC_T
    xT = jnp.transpose(x, (1, 2, 0))
    xT2 = xT.reshape(T * V, B)
    comb2 = -((jnp.arange(V, dtype=jnp.int32) << 8) | alphabet_codes
              ).astype(jnp.float32).reshape(V, 1)

    sc_out = _sc_decode_call(xT2, alphabet_codes, V, B, TC_T, sc_t)
    tc_out = _tc_decode_call(xT, comb2, T, V, B)
    oT = jnp.concatenate([tc_out, sc_out.reshape(sc_t, B)], axis=0)
    return jnp.transpose(oT)


# speedup vs baseline: 1.2781x; 1.0505x over previous
import functools

import jax
import jax.numpy as jnp
from jax import lax
from jax.experimental import pallas as pl
from jax.experimental.pallas import tpu as pltpu
from jax.experimental.pallas import tpu_sc as plsc

NC = 2
NS = 16
NW = NC * NS
LANES = 16
TC_T = 112
TC_TB = 8
TC_BB = 512
SC_BB = 512
NBUF = 2


def _sc_decode_call(xT2, alphabet_codes, V, B, t0, sc_t):
    nchunks = sc_t * (B // SC_BB)
    per_w = nchunks // NW

    mesh = plsc.VectorSubcoreMesh(
        core_axis_name="c", subcore_axis_name="s",
        num_cores=NC, num_subcores=NS)

    @functools.partial(
        pl.kernel,
        out_type=jax.ShapeDtypeStruct((sc_t * B,), jnp.int32),
        mesh=mesh,
        scratch_types=(
            [pltpu.VMEM((V, SC_BB), jnp.float32) for _ in range(NBUF)]
            + [pltpu.VMEM((SC_BB,), jnp.int32),
               pltpu.VMEM((V,), jnp.int32)]
            + [pltpu.SemaphoreType.DMA for _ in range(NBUF)]
        ),
        compiler_params=pltpu.CompilerParams(
            needs_layout_passes=False, use_tc_tiling_on_sc=True),
    )
    def sc_decode(x_hbm, alpha_hbm, out_hbm, *refs):
        bufs = refs[:NBUF]
        obuf, alpha_v = refs[NBUF], refs[NBUF + 1]
        sems = refs[NBUF + 2:]

        wid = lax.axis_index("s") * NC + lax.axis_index("c")
        k0 = wid * per_w
        ncol = B // SC_BB

        def in_slice(k):
            t = t0 + k // ncol
            b0 = (k % ncol) * SC_BB
            return x_hbm.at[pl.ds(t * V, V), pl.ds(b0, SC_BB)]

        pltpu.sync_copy(alpha_hbm, alpha_v)
        for b in range(NBUF):
            pltpu.async_copy(in_slice(k0 + b), bufs[b], sems[b])

        def chunk_body(k, b):
            pltpu.make_async_copy(in_slice(k), bufs[b], sems[b]).wait()
            bb = bufs[b]

            def group(gr, carry):
                c0 = gr * LANES
                ms, idxs = [], []
                for j in range(8):
                    base_c = j * 8
                    m = bb[base_c, pl.ds(c0, LANES)]
                    idx = jnp.full((LANES,), base_c, jnp.int32)
                    for q in range(1, 8):
                        c = base_c + q
                        v = bb[c, pl.ds(c0, LANES)]
                        upd = v > m
                        m = jnp.where(upd, v, m)
                        idx = jnp.where(upd, c, idx)
                    ms.append(m)
                    idxs.append(idx)
                while len(ms) > 1:
                    nm, ni = [], []
                    for j in range(0, len(ms), 2):
                        upd = ms[j + 1] > ms[j]
                        nm.append(jnp.where(upd, ms[j + 1], ms[j]))
                        ni.append(jnp.where(upd, idxs[j + 1], idxs[j]))
                    ms, idxs = nm, ni
                obuf[pl.ds(c0, LANES)] = plsc.load_gather(alpha_v, [idxs[0]])
                return carry

            lax.fori_loop(0, SC_BB // LANES, group, 0)

            t = t0 + k // ncol
            b0 = (k % ncol) * SC_BB
            pltpu.sync_copy(
                obuf, out_hbm.at[pl.ds((t - t0) * B + b0, SC_BB)])

            nxt = k + NBUF

            @pl.when(nxt < k0 + per_w)
            def _():
                pltpu.async_copy(in_slice(nxt), bufs[b], sems[b])

        def ring_body(i, carry):
            for b in range(NBUF):
                chunk_body(k0 + i * NBUF + b, b)
            return carry

        lax.fori_loop(0, per_w // NBUF, ring_body, 0)

    return sc_decode(xT2, alphabet_codes)


def _tc_decode_call(xT, comb2, T, V, B):
    def tc_kernel(comb_ref, x_ref, o_ref):
        xb = x_ref[...]
        m = jnp.max(xb, axis=1, keepdims=True)
        keyf = jnp.where(xb == m, comb_ref[...][None, :, :], -jnp.inf)
        best = jnp.max(keyf, axis=1)
        o_ref[...] = (-best).astype(jnp.int32) & 255

    return pl.pallas_call(
        tc_kernel,
        grid=(TC_T // TC_TB, B // TC_BB),
        in_specs=[
            pl.BlockSpec((V, 1), lambda i, j: (0, 0)),
            pl.BlockSpec((TC_TB, V, TC_BB), lambda i, j: (i, 0, j)),
        ],
        out_specs=pl.BlockSpec((TC_TB, TC_BB), lambda i, j: (i, j)),
        out_shape=jax.ShapeDtypeStruct((TC_T, B), jnp.int32),
        compiler_params=pltpu.CompilerParams(
            dimension_semantics=("parallel", "parallel")),
    )(comb2, xT)


def kernel(x, alphabet_codes):
    B, T, V = x.shape
    sc_t = T - TC_T
    xT = jnp.transpose(x, (1, 2, 0))
    xT2 = xT.reshape(T * V, B)
    comb2 = -((jnp.arange(V, dtype=jnp.int32) << 8) | alphabet_codes
              ).astype(jnp.float32).reshape(V, 1)

    sc_out = _sc_decode_call(xT2, alphabet_codes, V, B, TC_T, sc_t)
    tc_out = _tc_decode_call(xT, comb2, T, V, B)
    oT = jnp.concatenate([tc_out, sc_out.reshape(sc_t, B)], axis=0)
    return jnp.transpose(oT)


# speedup vs baseline: 1.3364x; 1.0456x over previous
import functools

import jax
import jax.numpy as jnp
from jax import lax
from jax.experimental import pallas as pl
from jax.experimental.pallas import tpu as pltpu
from jax.experimental.pallas import tpu_sc as plsc

NC = 2
NS = 16
NW = NC * NS
LANES = 16
TC_T = 104
TC_TB = 8
TC_BB = 512
SC_BB = 512
NBUF = 2


def _sc_decode_call(xT2, alphabet_codes, V, B, t0, sc_t):
    nchunks = sc_t * (B // SC_BB)
    per_w = nchunks // NW

    mesh = plsc.VectorSubcoreMesh(
        core_axis_name="c", subcore_axis_name="s",
        num_cores=NC, num_subcores=NS)

    @functools.partial(
        pl.kernel,
        out_type=jax.ShapeDtypeStruct((sc_t * B,), jnp.int32),
        mesh=mesh,
        scratch_types=(
            [pltpu.VMEM((V, SC_BB), jnp.float32) for _ in range(NBUF)]
            + [pltpu.VMEM((SC_BB,), jnp.int32),
               pltpu.VMEM((V,), jnp.int32)]
            + [pltpu.SemaphoreType.DMA for _ in range(NBUF)]
        ),
        compiler_params=pltpu.CompilerParams(
            needs_layout_passes=False, use_tc_tiling_on_sc=True),
    )
    def sc_decode(x_hbm, alpha_hbm, out_hbm, *refs):
        bufs = refs[:NBUF]
        obuf, alpha_v = refs[NBUF], refs[NBUF + 1]
        sems = refs[NBUF + 2:]

        wid = lax.axis_index("s") * NC + lax.axis_index("c")
        k0 = wid * per_w
        ncol = B // SC_BB

        def in_slice(k):
            t = t0 + k // ncol
            b0 = (k % ncol) * SC_BB
            return x_hbm.at[pl.ds(t * V, V), pl.ds(b0, SC_BB)]

        pltpu.sync_copy(alpha_hbm, alpha_v)
        for b in range(NBUF):
            pltpu.async_copy(in_slice(k0 + b), bufs[b], sems[b])

        def chunk_body(k, b):
            pltpu.make_async_copy(in_slice(k), bufs[b], sems[b]).wait()
            bb = bufs[b]

            def group(gr, carry):
                c0 = gr * LANES
                ms, idxs = [], []
                for j in range(8):
                    base_c = j * 8
                    m = bb[base_c, pl.ds(c0, LANES)]
                    idx = jnp.full((LANES,), base_c, jnp.int32)
                    for q in range(1, 8):
                        c = base_c + q
                        v = bb[c, pl.ds(c0, LANES)]
                        upd = v > m
                        m = jnp.where(upd, v, m)
                        idx = jnp.where(upd, c, idx)
                    ms.append(m)
                    idxs.append(idx)
                while len(ms) > 1:
                    nm, ni = [], []
                    for j in range(0, len(ms), 2):
                        upd = ms[j + 1] > ms[j]
                        nm.append(jnp.where(upd, ms[j + 1], ms[j]))
                        ni.append(jnp.where(upd, idxs[j + 1], idxs[j]))
                    ms, idxs = nm, ni
                obuf[pl.ds(c0, LANES)] = plsc.load_gather(alpha_v, [idxs[0]])
                return carry

            lax.fori_loop(0, SC_BB // LANES, group, 0)

            t = t0 + k // ncol
            b0 = (k % ncol) * SC_BB
            pltpu.sync_copy(
                obuf, out_hbm.at[pl.ds((t - t0) * B + b0, SC_BB)])

            nxt = k + NBUF

            @pl.when(nxt < k0 + per_w)
            def _():
                pltpu.async_copy(in_slice(nxt), bufs[b], sems[b])

        def ring_body(i, carry):
            for b in range(NBUF):
                chunk_body(k0 + i * NBUF + b, b)
            return carry

        lax.fori_loop(0, per_w // NBUF, ring_body, 0)

    return sc_decode(xT2, alphabet_codes)


def _tc_decode_call(xT, comb2, T, V, B):
    def tc_kernel(comb_ref, x_ref, o_ref):
        xb = x_ref[...]
        m = jnp.max(xb, axis=1, keepdims=True)
        keyf = jnp.where(xb == m, comb_ref[...][None, :, :], -jnp.inf)
        best = jnp.max(keyf, axis=1)
        o_ref[...] = (-best).astype(jnp.int32) & 255

    return pl.pallas_call(
        tc_kernel,
        grid=(TC_T // TC_TB, B // TC_BB),
        in_specs=[
            pl.BlockSpec((V, 1), lambda i, j: (0, 0)),
            pl.BlockSpec((TC_TB, V, TC_BB), lambda i, j: (i, 0, j)),
        ],
        out_specs=pl.BlockSpec((TC_TB, TC_BB), lambda i, j: (i, j)),
        out_shape=jax.ShapeDtypeStruct((TC_T, B), jnp.int32),
        compiler_params=pltpu.CompilerParams(
            dimension_semantics=("parallel", "parallel")),
    )(comb2, xT)


def kernel(x, alphabet_codes):
    B, T, V = x.shape
    sc_t = T - TC_T
    xT = jnp.transpose(x, (1, 2, 0))
    xT2 = xT.reshape(T * V, B)
    comb2 = -((jnp.arange(V, dtype=jnp.int32) << 8) | alphabet_codes
              ).astype(jnp.float32).reshape(V, 1)

    sc_out = _sc_decode_call(xT2, alphabet_codes, V, B, TC_T, sc_t)
    tc_out = _tc_decode_call(xT, comb2, T, V, B)
    oT = jnp.concatenate([tc_out, sc_out.reshape(sc_t, B)], axis=0)
    return jnp.transpose(oT)


# speedup vs baseline: 1.3935x; 1.0427x over previous
import functools

import jax
import jax.numpy as jnp
from jax import lax
from jax.experimental import pallas as pl
from jax.experimental.pallas import tpu as pltpu
from jax.experimental.pallas import tpu_sc as plsc

NC = 2
NS = 16
NW = NC * NS
LANES = 16
TC_T = 96
TC_TB = 8
TC_BB = 512
SC_BB = 512
NBUF = 2


def _sc_decode_call(xT2, alphabet_codes, V, B, t0, sc_t):
    nchunks = sc_t * (B // SC_BB)
    per_w = nchunks // NW

    mesh = plsc.VectorSubcoreMesh(
        core_axis_name="c", subcore_axis_name="s",
        num_cores=NC, num_subcores=NS)

    @functools.partial(
        pl.kernel,
        out_type=jax.ShapeDtypeStruct((sc_t * B,), jnp.int32),
        mesh=mesh,
        scratch_types=(
            [pltpu.VMEM((V, SC_BB), jnp.float32) for _ in range(NBUF)]
            + [pltpu.VMEM((SC_BB,), jnp.int32),
               pltpu.VMEM((V,), jnp.int32)]
            + [pltpu.SemaphoreType.DMA for _ in range(NBUF)]
        ),
        compiler_params=pltpu.CompilerParams(
            needs_layout_passes=False, use_tc_tiling_on_sc=True),
    )
    def sc_decode(x_hbm, alpha_hbm, out_hbm, *refs):
        bufs = refs[:NBUF]
        obuf, alpha_v = refs[NBUF], refs[NBUF + 1]
        sems = refs[NBUF + 2:]

        wid = lax.axis_index("s") * NC + lax.axis_index("c")
        k0 = wid * per_w
        ncol = B // SC_BB

        def in_slice(k):
            t = t0 + k // ncol
            b0 = (k % ncol) * SC_BB
            return x_hbm.at[pl.ds(t * V, V), pl.ds(b0, SC_BB)]

        pltpu.sync_copy(alpha_hbm, alpha_v)
        for b in range(NBUF):
            pltpu.async_copy(in_slice(k0 + b), bufs[b], sems[b])

        def chunk_body(k, b):
            pltpu.make_async_copy(in_slice(k), bufs[b], sems[b]).wait()
            bb = bufs[b]

            def group(gr, carry):
                c0 = gr * LANES
                ms, idxs = [], []
                for j in range(8):
                    base_c = j * 8
                    m = bb[base_c, pl.ds(c0, LANES)]
                    idx = jnp.full((LANES,), base_c, jnp.int32)
                    for q in range(1, 8):
                        c = base_c + q
                        v = bb[c, pl.ds(c0, LANES)]
                        upd = v > m
                        m = jnp.where(upd, v, m)
                        idx = jnp.where(upd, c, idx)
                    ms.append(m)
                    idxs.append(idx)
                while len(ms) > 1:
                    nm, ni = [], []
                    for j in range(0, len(ms), 2):
                        upd = ms[j + 1] > ms[j]
                        nm.append(jnp.where(upd, ms[j + 1], ms[j]))
                        ni.append(jnp.where(upd, idxs[j + 1], idxs[j]))
                    ms, idxs = nm, ni
                obuf[pl.ds(c0, LANES)] = plsc.load_gather(alpha_v, [idxs[0]])
                return carry

            lax.fori_loop(0, SC_BB // LANES, group, 0)

            t = t0 + k // ncol
            b0 = (k % ncol) * SC_BB
            pltpu.sync_copy(
                obuf, out_hbm.at[pl.ds((t - t0) * B + b0, SC_BB)])

            nxt = k + NBUF

            @pl.when(nxt < k0 + per_w)
            def _():
                pltpu.async_copy(in_slice(nxt), bufs[b], sems[b])

        def ring_body(i, carry):
            for b in range(NBUF):
                chunk_body(k0 + i * NBUF + b, b)
            return carry

        lax.fori_loop(0, per_w // NBUF, ring_body, 0)

    return sc_decode(xT2, alphabet_codes)


def _tc_decode_call(xT, comb2, T, V, B):
    def tc_kernel(comb_ref, x_ref, o_ref):
        xb = x_ref[...]
        m = jnp.max(xb, axis=1, keepdims=True)
        keyf = jnp.where(xb == m, comb_ref[...][None, :, :], -jnp.inf)
        best = jnp.max(keyf, axis=1)
        o_ref[...] = (-best).astype(jnp.int32) & 255

    return pl.pallas_call(
        tc_kernel,
        grid=(TC_T // TC_TB, B // TC_BB),
        in_specs=[
            pl.BlockSpec((V, 1), lambda i, j: (0, 0)),
            pl.BlockSpec((TC_TB, V, TC_BB), lambda i, j: (i, 0, j)),
        ],
        out_specs=pl.BlockSpec((TC_TB, TC_BB), lambda i, j: (i, j)),
        out_shape=jax.ShapeDtypeStruct((TC_T, B), jnp.int32),
        compiler_params=pltpu.CompilerParams(
            dimension_semantics=("parallel", "parallel")),
    )(comb2, xT)


def kernel(x, alphabet_codes):
    B, T, V = x.shape
    sc_t = T - TC_T
    xT = jnp.transpose(x, (1, 2, 0))
    xT2 = xT.reshape(T * V, B)
    comb2 = -((jnp.arange(V, dtype=jnp.int32) << 8) | alphabet_codes
              ).astype(jnp.float32).reshape(V, 1)

    sc_out = _sc_decode_call(xT2, alphabet_codes, V, B, TC_T, sc_t)
    tc_out = _tc_decode_call(xT, comb2, T, V, B)
    oT = jnp.concatenate([tc_out, sc_out.reshape(sc_t, B)], axis=0)
    return jnp.transpose(oT)


# speedup vs baseline: 1.4643x; 1.0508x over previous
import functools

import jax
import jax.numpy as jnp
from jax import lax
from jax.experimental import pallas as pl
from jax.experimental.pallas import tpu as pltpu
from jax.experimental.pallas import tpu_sc as plsc

NC = 2
NS = 16
NW = NC * NS
LANES = 16
TC_T = 88
TC_TB = 8
TC_BB = 512
SC_BB = 512
NBUF = 2


def _sc_decode_call(xT2, alphabet_codes, V, B, t0, sc_t):
    nchunks = sc_t * (B // SC_BB)
    per_w = nchunks // NW

    mesh = plsc.VectorSubcoreMesh(
        core_axis_name="c", subcore_axis_name="s",
        num_cores=NC, num_subcores=NS)

    @functools.partial(
        pl.kernel,
        out_type=jax.ShapeDtypeStruct((sc_t * B,), jnp.int32),
        mesh=mesh,
        scratch_types=(
            [pltpu.VMEM((V, SC_BB), jnp.float32) for _ in range(NBUF)]
            + [pltpu.VMEM((SC_BB,), jnp.int32),
               pltpu.VMEM((V,), jnp.int32)]
            + [pltpu.SemaphoreType.DMA for _ in range(NBUF)]
        ),
        compiler_params=pltpu.CompilerParams(
            needs_layout_passes=False, use_tc_tiling_on_sc=True),
    )
    def sc_decode(x_hbm, alpha_hbm, out_hbm, *refs):
        bufs = refs[:NBUF]
        obuf, alpha_v = refs[NBUF], refs[NBUF + 1]
        sems = refs[NBUF + 2:]

        wid = lax.axis_index("s") * NC + lax.axis_index("c")
        k0 = wid * per_w
        ncol = B // SC_BB

        def in_slice(k):
            t = t0 + k // ncol
            b0 = (k % ncol) * SC_BB
            return x_hbm.at[pl.ds(t * V, V), pl.ds(b0, SC_BB)]

        pltpu.sync_copy(alpha_hbm, alpha_v)
        for b in range(NBUF):
            pltpu.async_copy(in_slice(k0 + b), bufs[b], sems[b])

        def chunk_body(k, b):
            pltpu.make_async_copy(in_slice(k), bufs[b], sems[b]).wait()
            bb = bufs[b]

            def group(gr, carry):
                c0 = gr * LANES
                ms, idxs = [], []
                for j in range(8):
                    base_c = j * 8
                    m = bb[base_c, pl.ds(c0, LANES)]
                    idx = jnp.full((LANES,), base_c, jnp.int32)
                    for q in range(1, 8):
                        c = base_c + q
                        v = bb[c, pl.ds(c0, LANES)]
                        upd = v > m
                        m = jnp.where(upd, v, m)
                        idx = jnp.where(upd, c, idx)
                    ms.append(m)
                    idxs.append(idx)
                while len(ms) > 1:
                    nm, ni = [], []
                    for j in range(0, len(ms), 2):
                        upd = ms[j + 1] > ms[j]
                        nm.append(jnp.where(upd, ms[j + 1], ms[j]))
                        ni.append(jnp.where(upd, idxs[j + 1], idxs[j]))
                    ms, idxs = nm, ni
                obuf[pl.ds(c0, LANES)] = plsc.load_gather(alpha_v, [idxs[0]])
                return carry

            lax.fori_loop(0, SC_BB // LANES, group, 0)

            t = t0 + k // ncol
            b0 = (k % ncol) * SC_BB
            pltpu.sync_copy(
                obuf, out_hbm.at[pl.ds((t - t0) * B + b0, SC_BB)])

            nxt = k + NBUF

            @pl.when(nxt < k0 + per_w)
            def _():
                pltpu.async_copy(in_slice(nxt), bufs[b], sems[b])

        def ring_body(i, carry):
            for b in range(NBUF):
                chunk_body(k0 + i * NBUF + b, b)
            return carry

        lax.fori_loop(0, per_w // NBUF, ring_body, 0)

    return sc_decode(xT2, alphabet_codes)


def _tc_decode_call(xT, comb2, T, V, B):
    def tc_kernel(comb_ref, x_ref, o_ref):
        xb = x_ref[...]
        m = jnp.max(xb, axis=1, keepdims=True)
        keyf = jnp.where(xb == m, comb_ref[...][None, :, :], -jnp.inf)
        best = jnp.max(keyf, axis=1)
        o_ref[...] = (-best).astype(jnp.int32) & 255

    return pl.pallas_call(
        tc_kernel,
        grid=(TC_T // TC_TB, B // TC_BB),
        in_specs=[
            pl.BlockSpec((V, 1), lambda i, j: (0, 0)),
            pl.BlockSpec((TC_TB, V, TC_BB), lambda i, j: (i, 0, j)),
        ],
        out_specs=pl.BlockSpec((TC_TB, TC_BB), lambda i, j: (i, j)),
        out_shape=jax.ShapeDtypeStruct((TC_T, B), jnp.int32),
        compiler_params=pltpu.CompilerParams(
            dimension_semantics=("parallel", "parallel")),
    )(comb2, xT)


def kernel(x, alphabet_codes):
    B, T, V = x.shape
    sc_t = T - TC_T
    xT = jnp.transpose(x, (1, 2, 0))
    xT2 = xT.reshape(T * V, B)
    comb2 = -((jnp.arange(V, dtype=jnp.int32) << 8) | alphabet_codes
              ).astype(jnp.float32).reshape(V, 1)

    sc_out = _sc_decode_call(xT2, alphabet_codes, V, B, TC_T, sc_t)
    tc_out = _tc_decode_call(xT, comb2, T, V, B)
    oT = jnp.concatenate([tc_out, sc_out.reshape(sc_t, B)], axis=0)
    return jnp.transpose(oT)


# speedup vs baseline: 1.5441x; 1.0545x over previous
import functools

import jax
import jax.numpy as jnp
from jax import lax
from jax.experimental import pallas as pl
from jax.experimental.pallas import tpu as pltpu
from jax.experimental.pallas import tpu_sc as plsc

NC = 2
NS = 16
NW = NC * NS
LANES = 16
TC_T = 80
TC_TB = 8
TC_BB = 512
SC_BB = 512
NBUF = 2


def _sc_decode_call(xT2, alphabet_codes, V, B, t0, sc_t):
    nchunks = sc_t * (B // SC_BB)
    per_w = nchunks // NW

    mesh = plsc.VectorSubcoreMesh(
        core_axis_name="c", subcore_axis_name="s",
        num_cores=NC, num_subcores=NS)

    @functools.partial(
        pl.kernel,
        out_type=jax.ShapeDtypeStruct((sc_t * B,), jnp.int32),
        mesh=mesh,
        scratch_types=(
            [pltpu.VMEM((V, SC_BB), jnp.float32) for _ in range(NBUF)]
            + [pltpu.VMEM((SC_BB,), jnp.int32),
               pltpu.VMEM((V,), jnp.int32)]
            + [pltpu.SemaphoreType.DMA for _ in range(NBUF)]
        ),
        compiler_params=pltpu.CompilerParams(
            needs_layout_passes=False, use_tc_tiling_on_sc=True),
    )
    def sc_decode(x_hbm, alpha_hbm, out_hbm, *refs):
        bufs = refs[:NBUF]
        obuf, alpha_v = refs[NBUF], refs[NBUF + 1]
        sems = refs[NBUF + 2:]

        wid = lax.axis_index("s") * NC + lax.axis_index("c")
        k0 = wid * per_w
        ncol = B // SC_BB

        def in_slice(k):
            t = t0 + k // ncol
            b0 = (k % ncol) * SC_BB
            return x_hbm.at[pl.ds(t * V, V), pl.ds(b0, SC_BB)]

        pltpu.sync_copy(alpha_hbm, alpha_v)
        for b in range(NBUF):
            pltpu.async_copy(in_slice(k0 + b), bufs[b], sems[b])

        def chunk_body(k, b):
            pltpu.make_async_copy(in_slice(k), bufs[b], sems[b]).wait()
            bb = bufs[b]

            def group(gr, carry):
                c0 = gr * LANES
                ms, idxs = [], []
                for j in range(8):
                    base_c = j * 8
                    m = bb[base_c, pl.ds(c0, LANES)]
                    idx = jnp.full((LANES,), base_c, jnp.int32)
                    for q in range(1, 8):
                        c = base_c + q
                        v = bb[c, pl.ds(c0, LANES)]
                        upd = v > m
                        m = jnp.where(upd, v, m)
                        idx = jnp.where(upd, c, idx)
                    ms.append(m)
                    idxs.append(idx)
                while len(ms) > 1:
                    nm, ni = [], []
                    for j in range(0, len(ms), 2):
                        upd = ms[j + 1] > ms[j]
                        nm.append(jnp.where(upd, ms[j + 1], ms[j]))
                        ni.append(jnp.where(upd, idxs[j + 1], idxs[j]))
                    ms, idxs = nm, ni
                obuf[pl.ds(c0, LANES)] = plsc.load_gather(alpha_v, [idxs[0]])
                return carry

            lax.fori_loop(0, SC_BB // LANES, group, 0)

            t = t0 + k // ncol
            b0 = (k % ncol) * SC_BB
            pltpu.sync_copy(
                obuf, out_hbm.at[pl.ds((t - t0) * B + b0, SC_BB)])

            nxt = k + NBUF

            @pl.when(nxt < k0 + per_w)
            def _():
                pltpu.async_copy(in_slice(nxt), bufs[b], sems[b])

        def ring_body(i, carry):
            for b in range(NBUF):
                chunk_body(k0 + i * NBUF + b, b)
            return carry

        lax.fori_loop(0, per_w // NBUF, ring_body, 0)

    return sc_decode(xT2, alphabet_codes)


def _tc_decode_call(xT, comb2, T, V, B):
    def tc_kernel(comb_ref, x_ref, o_ref):
        xb = x_ref[...]
        m = jnp.max(xb, axis=1, keepdims=True)
        keyf = jnp.where(xb == m, comb_ref[...][None, :, :], -jnp.inf)
        best = jnp.max(keyf, axis=1)
        o_ref[...] = (-best).astype(jnp.int32) & 255

    return pl.pallas_call(
        tc_kernel,
        grid=(TC_T // TC_TB, B // TC_BB),
        in_specs=[
            pl.BlockSpec((V, 1), lambda i, j: (0, 0)),
            pl.BlockSpec((TC_TB, V, TC_BB), lambda i, j: (i, 0, j)),
        ],
        out_specs=pl.BlockSpec((TC_TB, TC_BB), lambda i, j: (i, j)),
        out_shape=jax.ShapeDtypeStruct((TC_T, B), jnp.int32),
        compiler_params=pltpu.CompilerParams(
            dimension_semantics=("parallel", "parallel")),
    )(comb2, xT)


def kernel(x, alphabet_codes):
    B, T, V = x.shape
    sc_t = T - TC_T
    xT = jnp.transpose(x, (1, 2, 0))
    xT2 = xT.reshape(T * V, B)
    comb2 = -((jnp.arange(V, dtype=jnp.int32) << 8) | alphabet_codes
              ).astype(jnp.float32).reshape(V, 1)

    sc_out = _sc_decode_call(xT2, alphabet_codes, V, B, TC_T, sc_t)
    tc_out = _tc_decode_call(xT, comb2, T, V, B)
    oT = jnp.concatenate([tc_out, sc_out.reshape(sc_t, B)], axis=0)
    return jnp.transpose(oT)


# speedup vs baseline: 1.5446x; 1.0003x over previous
import functools

import jax
import jax.numpy as jnp
from jax import lax
from jax.experimental import pallas as pl
from jax.experimental.pallas import tpu as pltpu
from jax.experimental.pallas import tpu_sc as plsc

NC = 2
NS = 16
NW = NC * NS
LANES = 16
TC_T = 64
TC_TB = 8
TC_BB = 512
SC_BB = 512
NBUF = 2


def _sc_decode_call(xT2, alphabet_codes, V, B, t0, sc_t):
    nchunks = sc_t * (B // SC_BB)
    per_w = nchunks // NW

    mesh = plsc.VectorSubcoreMesh(
        core_axis_name="c", subcore_axis_name="s",
        num_cores=NC, num_subcores=NS)

    @functools.partial(
        pl.kernel,
        out_type=jax.ShapeDtypeStruct((sc_t * B,), jnp.int32),
        mesh=mesh,
        scratch_types=(
            [pltpu.VMEM((V, SC_BB), jnp.float32) for _ in range(NBUF)]
            + [pltpu.VMEM((SC_BB,), jnp.int32),
               pltpu.VMEM((V,), jnp.int32)]
            + [pltpu.SemaphoreType.DMA for _ in range(NBUF)]
        ),
        compiler_params=pltpu.CompilerParams(
            needs_layout_passes=False, use_tc_tiling_on_sc=True),
    )
    def sc_decode(x_hbm, alpha_hbm, out_hbm, *refs):
        bufs = refs[:NBUF]
        obuf, alpha_v = refs[NBUF], refs[NBUF + 1]
        sems = refs[NBUF + 2:]

        wid = lax.axis_index("s") * NC + lax.axis_index("c")
        k0 = wid * per_w
        ncol = B // SC_BB

        def in_slice(k):
            t = t0 + k // ncol
            b0 = (k % ncol) * SC_BB
            return x_hbm.at[pl.ds(t * V, V), pl.ds(b0, SC_BB)]

        pltpu.sync_copy(alpha_hbm, alpha_v)
        for b in range(NBUF):
            pltpu.async_copy(in_slice(k0 + b), bufs[b], sems[b])

        def chunk_body(k, b):
            pltpu.make_async_copy(in_slice(k), bufs[b], sems[b]).wait()
            bb = bufs[b]

            def group(gr, carry):
                c0 = gr * LANES
                ms, idxs = [], []
                for j in range(8):
                    base_c = j * 8
                    m = bb[base_c, pl.ds(c0, LANES)]
                    idx = jnp.full((LANES,), base_c, jnp.int32)
                    for q in range(1, 8):
                        c = base_c + q
                        v = bb[c, pl.ds(c0, LANES)]
                        upd = v > m
                        m = jnp.where(upd, v, m)
                        idx = jnp.where(upd, c, idx)
                    ms.append(m)
                    idxs.append(idx)
                while len(ms) > 1:
                    nm, ni = [], []
                    for j in range(0, len(ms), 2):
                        upd = ms[j + 1] > ms[j]
                        nm.append(jnp.where(upd, ms[j + 1], ms[j]))
                        ni.append(jnp.where(upd, idxs[j + 1], idxs[j]))
                    ms, idxs = nm, ni
                obuf[pl.ds(c0, LANES)] = plsc.load_gather(alpha_v, [idxs[0]])
                return carry

            lax.fori_loop(0, SC_BB // LANES, group, 0)

            t = t0 + k // ncol
            b0 = (k % ncol) * SC_BB
            pltpu.sync_copy(
                obuf, out_hbm.at[pl.ds((t - t0) * B + b0, SC_BB)])

            nxt = k + NBUF

            @pl.when(nxt < k0 + per_w)
            def _():
                pltpu.async_copy(in_slice(nxt), bufs[b], sems[b])

        def ring_body(i, carry):
            for b in range(NBUF):
                chunk_body(k0 + i * NBUF + b, b)
            return carry

        lax.fori_loop(0, per_w // NBUF, ring_body, 0)

    return sc_decode(xT2, alphabet_codes)


def _tc_decode_call(xT, comb2, T, V, B):
    def tc_kernel(comb_ref, x_ref, o_ref):
        xb = x_ref[...]
        m = jnp.max(xb, axis=1, keepdims=True)
        keyf = jnp.where(xb == m, comb_ref[...][None, :, :], -jnp.inf)
        best = jnp.max(keyf, axis=1)
        o_ref[...] = (-best).astype(jnp.int32) & 255

    return pl.pallas_call(
        tc_kernel,
        grid=(TC_T // TC_TB, B // TC_BB),
        in_specs=[
            pl.BlockSpec((V, 1), lambda i, j: (0, 0)),
            pl.BlockSpec((TC_TB, V, TC_BB), lambda i, j: (i, 0, j)),
        ],
        out_specs=pl.BlockSpec((TC_TB, TC_BB), lambda i, j: (i, j)),
        out_shape=jax.ShapeDtypeStruct((TC_T, B), jnp.int32),
        compiler_params=pltpu.CompilerParams(
            dimension_semantics=("parallel", "parallel")),
    )(comb2, xT)


def kernel(x, alphabet_codes):
    B, T, V = x.shape
    sc_t = T - TC_T
    xT = jnp.transpose(x, (1, 2, 0))
    xT2 = xT.reshape(T * V, B)
    comb2 = -((jnp.arange(V, dtype=jnp.int32) << 8) | alphabet_codes
              ).astype(jnp.float32).reshape(V, 1)

    sc_out = _sc_decode_call(xT2, alphabet_codes, V, B, TC_T, sc_t)
    tc_out = _tc_decode_call(xT, comb2, T, V, B)
    oT = jnp.concatenate([tc_out, sc_out.reshape(sc_t, B)], axis=0)
    return jnp.transpose(oT)
